# baseline (speedup 1.0000x reference)
import functools

import jax
import jax.numpy as jnp
from jax import lax
from jax.experimental import pallas as pl
from jax.experimental.pallas import tpu as pltpu
from jax.experimental.pallas import tpu_sc as plsc

_PERCENT_ON = 0.1
_BOOST_STRENGTH = 1.0
_L = 16


def _suffix_step(h, base, acc_above, need):
    rc = lax.rev(plsc.cumsum(lax.rev(h, (0,))), (0,))
    s = acc_above + rc
    lane_b = lax.iota(jnp.int32, _L) + base
    cand = jnp.max(jnp.where(s >= need, lane_b, -1))
    n_above = acc_above + jnp.sum(jnp.where(lane_b > cand, h, 0))
    return cand, n_above


def _scan_hist(hist_ref, nbins, need):
    def step(i, carry):
        acc_above, b_star, n_above = carry
        c = (nbins // _L - 1) - i
        h = hist_ref[pl.ds(c * _L, _L)]
        cand, n_above_new = _suffix_step(h, c * _L, acc_above, need)
        found_now = (b_star < 0) & (cand >= 0)
        b_star = jnp.where(found_now, cand, b_star)
        n_above = jnp.where(found_now, n_above_new, n_above)
        acc_above = acc_above + jnp.sum(h)
        return acc_above, b_star, n_above

    init = (jnp.int32(0), jnp.int32(-1), jnp.int32(0))
    _, b_star, n_above = lax.fori_loop(0, nbins // _L, step, init)
    return b_star, n_above


def _select2(hist_v, hh_v, need):
    hi, n_above_hi = _scan_hist(hh_v, 256, need)
    h = hist_v[pl.ds(hi * _L, _L)]
    return _suffix_step(h, hi * _L, n_above_hi, need)


def _fold_coarse(hist_v, hh_v):
    lane0 = lax.iota(jnp.int32, _L) == 0
    zeros = jnp.zeros((_L,), jnp.int32)

    @plsc.parallel_loop(0, 256, unroll=8)
    def _(j):
        s = jnp.sum(hist_v[pl.ds(j * _L, _L)])
        plsc.store_scatter(hh_v, [zeros + j], zeros + s, mask=lane0)


def _zero(ref, nwords):
    @plsc.parallel_loop(0, nwords // _L, unroll=8)
    def _(i):
        ref[pl.ds(i * _L, _L)] = jnp.zeros((_L,), jnp.int32)


def _i32(v):
    return lax.bitcast_convert_type(v, jnp.int32)


def _f32(v):
    return lax.bitcast_convert_type(v, jnp.float32)


def _row_passes(buf, bf_v, hist_v, hh_v, nvec, k):
    ones = jnp.full((_L,), 1, jnp.int32)

    _zero(hist_v, 4096)

    @plsc.parallel_loop(0, nvec, unroll=16)
    def _(i):
        ds = pl.ds(i * _L, _L)
        bits = _i32(buf[ds] * bf_v[ds])
        skey = jnp.where(bits >= 0, bits, bits ^ jnp.int32(0x7FFFFFFF))
        buf[ds] = _f32(skey)
        b1 = (skey >> 20) + 2048
        plsc.addupdate_scatter(hist_v, [b1], ones)

    _fold_coarse(hist_v, hh_v)
    b1s, n_above1 = _select2(hist_v, hh_v, jnp.int32(k))
    need2 = jnp.int32(k) - n_above1

    _zero(hist_v, 4096)

    @plsc.parallel_loop(0, nvec, unroll=16)
    def _(i):
        skey = _i32(buf[pl.ds(i * _L, _L)])
        m = ((skey >> 20) + 2048) == b1s
        b2 = (skey >> 8) & 0xFFF
        plsc.addupdate_scatter(hist_v, [b2], ones, mask=m)

    _fold_coarse(hist_v, hh_v)
    b2s, n_above2 = _select2(hist_v, hh_v, need2)
    need3 = need2 - n_above2

    _zero(hh_v, 256)

    @plsc.parallel_loop(0, nvec, unroll=16)
    def _(i):
        skey = _i32(buf[pl.ds(i * _L, _L)])
        m = (((skey >> 20) + 2048) == b1s) & (((skey >> 8) & 0xFFF) == b2s)
        plsc.addupdate_scatter(hh_v, [skey & 0xFF], ones, mask=m)

    b3s, _ = _scan_hist(hh_v, 256, need3)

    thr = ((b1s - 2048) << 20) | (b2s << 8) | b3s

    @plsc.parallel_loop(0, nvec, unroll=16)
    def _(i):
        ds = pl.ds(i * _L, _L)
        skey = _i32(buf[ds])
        bits = jnp.where(skey >= 0, skey, skey ^ jnp.int32(0x7FFFFFFF))
        xv = _f32(bits) / bf_v[ds]
        buf[ds] = jnp.where(skey >= thr, xv, jnp.float32(0.0))


def _sc_body(x_hbm, duty_hbm, out_hbm, bf_v, buf0, buf1, hist_v, hh_v,
             sem_in, sem_out, *, k):
    b, n = x_hbm.shape
    nvec = n // _L
    rows_per = b // 32
    wid = lax.axis_index("s") * 2 + lax.axis_index("c")
    td = jnp.float32(k / n)
    row0 = wid * rows_per

    bufs = [buf0, buf1]
    in_d = {0: pltpu.async_copy(x_hbm.at[row0], buf0, sem_in)}

    pltpu.sync_copy(duty_hbm, bf_v)

    @plsc.parallel_loop(0, nvec, unroll=16)
    def _(i):
        ds = pl.ds(i * _L, _L)
        bf_v[ds] = jnp.exp((td - bf_v[ds]) * jnp.float32(_BOOST_STRENGTH))

    out_d = {}
    for r in range(rows_per):
        cur = bufs[r % 2]
        in_d[r].wait()
        if r + 1 < rows_per:
            if r - 1 >= 0:
                out_d[r - 1].wait()
            in_d[r + 1] = pltpu.async_copy(
                x_hbm.at[row0 + r + 1], bufs[(r + 1) % 2], sem_in)
        _row_passes(cur, bf_v, hist_v, hh_v, nvec, k)
        out_d[r] = pltpu.async_copy(cur, out_hbm.at[row0 + r], sem_out)
    out_d[rows_per - 2].wait()
    out_d[rows_per - 1].wait()


@jax.jit
def kernel(x, duty_cycles):
    b, n = x.shape
    k = int(round(n * _PERCENT_ON))
    mesh = plsc.VectorSubcoreMesh(core_axis_name="c", subcore_axis_name="s")
    run = pl.kernel(
        functools.partial(_sc_body, k=k),
        out_type=jax.ShapeDtypeStruct((b, n), jnp.float32),
        mesh=mesh,
        compiler_params=pltpu.CompilerParams(needs_layout_passes=False),
        scratch_types=[
            pltpu.VMEM((n,), jnp.float32),
            pltpu.VMEM((n,), jnp.float32),
            pltpu.VMEM((n,), jnp.float32),
            pltpu.VMEM((4096,), jnp.int32),
            pltpu.VMEM((256,), jnp.int32),
            pltpu.SemaphoreType.DMA,
            pltpu.SemaphoreType.DMA,
        ],
    )
    return run(x, duty_cycles)


# speedup vs baseline: 1.4655x; 1.4655x over previous
import functools

import jax
import jax.numpy as jnp
from jax import lax
from jax.experimental import pallas as pl
from jax.experimental.pallas import tpu as pltpu
from jax.experimental.pallas import tpu_sc as plsc

_PERCENT_ON = 0.1
_BOOST_STRENGTH = 1.0
_L = 16


def _suffix_step(h, base, acc_above, need):
    rc = lax.rev(plsc.cumsum(lax.rev(h, (0,))), (0,))
    s = acc_above + rc
    lane_b = lax.iota(jnp.int32, _L) + base
    cand = jnp.max(jnp.where(s >= need, lane_b, -1))
    n_above = acc_above + jnp.sum(jnp.where(lane_b > cand, h, 0))
    return cand, n_above


def _scan_hist(hist_ref, nbins, need):
    def step(i, carry):
        acc_above, b_star, n_above = carry
        c = (nbins // _L - 1) - i
        h = hist_ref[pl.ds(c * _L, _L)]
        cand, n_above_new = _suffix_step(h, c * _L, acc_above, need)
        found_now = (b_star < 0) & (cand >= 0)
        b_star = jnp.where(found_now, cand, b_star)
        n_above = jnp.where(found_now, n_above_new, n_above)
        acc_above = acc_above + jnp.sum(h)
        return acc_above, b_star, n_above

    init = (jnp.int32(0), jnp.int32(-1), jnp.int32(0))
    _, b_star, n_above = lax.fori_loop(0, nbins // _L, step, init)
    return b_star, n_above


def _select2(hist_v, hh_v, need):
    hi, n_above_hi = _scan_hist(hh_v, 256, need)
    h = hist_v[pl.ds(hi * _L, _L)]
    return _suffix_step(h, hi * _L, n_above_hi, need)


def _fold_coarse(hist_v, hh_v):
    lane0 = lax.iota(jnp.int32, _L) == 0
    zeros = jnp.zeros((_L,), jnp.int32)

    @plsc.parallel_loop(0, 256, unroll=8)
    def _(j):
        s = jnp.sum(hist_v[pl.ds(j * _L, _L)])
        plsc.store_scatter(hh_v, [zeros + j], zeros + s, mask=lane0)


def _zero(ref, nwords):
    @plsc.parallel_loop(0, nwords // _L, unroll=8)
    def _(i):
        ref[pl.ds(i * _L, _L)] = jnp.zeros((_L,), jnp.int32)


def _i32(v):
    return lax.bitcast_convert_type(v, jnp.int32)


def _f32(v):
    return lax.bitcast_convert_type(v, jnp.float32)


def _row_passes(buf, bf_v, hist_v, hh_v, nvec, k):
    ones = jnp.full((_L,), 1, jnp.int32)

    _zero(hist_v, 4096)

    @plsc.parallel_loop(0, nvec, unroll=8)
    def _(i):
        ds = pl.ds(i * _L, _L)
        bits = _i32(buf[ds] * bf_v[ds])
        skey = jnp.where(bits >= 0, bits, bits ^ jnp.int32(0x7FFFFFFF))
        buf[ds] = _f32(skey)
        b1 = (skey >> 20) + 2048
        plsc.addupdate_scatter(hist_v, [b1], ones)

    _fold_coarse(hist_v, hh_v)
    b1s, n_above1 = _select2(hist_v, hh_v, jnp.int32(k))
    need2 = jnp.int32(k) - n_above1

    _zero(hist_v, 4096)

    @plsc.parallel_loop(0, nvec, unroll=8)
    def _(i):
        skey = _i32(buf[pl.ds(i * _L, _L)])
        m = ((skey >> 20) + 2048) == b1s
        b2 = (skey >> 8) & 0xFFF
        plsc.addupdate_scatter(hist_v, [b2], ones, mask=m)

    _fold_coarse(hist_v, hh_v)
    b2s, n_above2 = _select2(hist_v, hh_v, need2)
    need3 = need2 - n_above2

    _zero(hh_v, 256)

    @plsc.parallel_loop(0, nvec, unroll=8)
    def _(i):
        skey = _i32(buf[pl.ds(i * _L, _L)])
        m = (((skey >> 20) + 2048) == b1s) & (((skey >> 8) & 0xFFF) == b2s)
        plsc.addupdate_scatter(hh_v, [skey & 0xFF], ones, mask=m)

    b3s, _ = _scan_hist(hh_v, 256, need3)

    thr = ((b1s - 2048) << 20) | (b2s << 8) | b3s

    @plsc.parallel_loop(0, nvec, unroll=8)
    def _(i):
        ds = pl.ds(i * _L, _L)
        skey = _i32(buf[ds])
        bits = jnp.where(skey >= 0, skey, skey ^ jnp.int32(0x7FFFFFFF))
        xv = _f32(bits) / bf_v[ds]
        buf[ds] = jnp.where(skey >= thr, xv, jnp.float32(0.0))


def _sc_body(x_hbm, duty_hbm, out_hbm, bf_v, buf0, buf1, hist_v, hh_v,
             sem_in, sem_out, *, k):
    b, n = x_hbm.shape
    nvec = n // _L
    rows_per = b // 32
    wid = lax.axis_index("s") * 2 + lax.axis_index("c")
    td = jnp.float32(k / n)
    row0 = wid * rows_per

    bufs = [buf0, buf1]
    in_d = {0: pltpu.async_copy(x_hbm.at[row0], buf0, sem_in)}

    pltpu.sync_copy(duty_hbm, bf_v)

    @plsc.parallel_loop(0, nvec, unroll=8)
    def _(i):
        ds = pl.ds(i * _L, _L)
        bf_v[ds] = jnp.exp((td - bf_v[ds]) * jnp.float32(_BOOST_STRENGTH))

    out_d = {}
    for r in range(rows_per):
        cur = bufs[r % 2]
        in_d[r].wait()
        if r + 1 < rows_per:
            if r - 1 >= 0:
                out_d[r - 1].wait()
            in_d[r + 1] = pltpu.async_copy(
                x_hbm.at[row0 + r + 1], bufs[(r + 1) % 2], sem_in)
        _row_passes(cur, bf_v, hist_v, hh_v, nvec, k)
        out_d[r] = pltpu.async_copy(cur, out_hbm.at[row0 + r], sem_out)
    out_d[rows_per - 2].wait()
    out_d[rows_per - 1].wait()


@jax.jit
def kernel(x, duty_cycles):
    b, n = x.shape
    k = int(round(n * _PERCENT_ON))
    mesh = plsc.VectorSubcoreMesh(core_axis_name="c", subcore_axis_name="s")
    run = pl.kernel(
        functools.partial(_sc_body, k=k),
        out_type=jax.ShapeDtypeStruct((b, n), jnp.float32),
        mesh=mesh,
        compiler_params=pltpu.CompilerParams(needs_layout_passes=False),
        scratch_types=[
            pltpu.VMEM((n,), jnp.float32),
            pltpu.VMEM((n,), jnp.float32),
            pltpu.VMEM((n,), jnp.float32),
            pltpu.VMEM((4096,), jnp.int32),
            pltpu.VMEM((256,), jnp.int32),
            pltpu.SemaphoreType.DMA,
            pltpu.SemaphoreType.DMA,
        ],
    )
    return run(x, duty_cycles)
